# Initial kernel scaffold; baseline (speedup 1.0000x reference)
import jax
import jax.numpy as jnp
from jax.experimental import pallas as pl


def kernel(x, edge_index, adj_values, weight):
    raise NotImplementedError("write your pallas kernel here")


# speedup vs baseline: 2.9116x; 2.9116x over previous
import jax
import jax.numpy as jnp
from jax import lax
from jax.experimental import pallas as pl
from jax.experimental.pallas import tpu as pltpu
from jax.experimental.pallas import tpu_sc as plsc

N = 10000
NP = 10240
D = 128
E = 320000
NC = 2
NS = 16
NW = NC * NS
EPT = 10240
E_PAD = NW * EPT
CHUNK = 128
NCHUNK = EPT // CHUNK
RPT = NP // NS

_LANES = 16


def _sc_body(x_hbm, col_hbm, row_hbm, adj_hbm, zero_hbm, out_hbm,
             colv, rowv, adjv, rows, acc, gsem):
    c = lax.axis_index("c")
    s = lax.axis_index("s")
    wid = s * NC + c

    pltpu.sync_copy(zero_hbm, acc.at[pl.ds(s * RPT, RPT)])
    pltpu.sync_copy(col_hbm.at[wid], colv)
    pltpu.sync_copy(row_hbm.at[wid], rowv)
    pltpu.sync_copy(adj_hbm.at[wid], adjv)
    plsc.subcore_barrier()

    def chunk_body(k, carry):
        pltpu.async_copy(x_hbm.at[colv.at[k]], rows, gsem).wait()
        base = k * CHUNK
        for g in range(CHUNK // _LANES):
            a16 = adjv[pl.ds(base + g * _LANES, _LANES)]
            for j in range(_LANES):
                e = g * _LANES + j
                scale = jnp.full((_LANES,), a16[j], jnp.float32)
                for v in range(D // _LANES):
                    sl = pl.ds(v * _LANES, _LANES)
                    rows[e, sl] = rows[e, sl] * scale
        pltpu.sync_copy(rows, acc.at[rowv.at[k]], add=True)
        return carry

    lax.fori_loop(0, NCHUNK, chunk_body, 0)
    plsc.subcore_barrier()
    pltpu.sync_copy(acc.at[pl.ds(s * RPT, RPT)],
                    out_hbm.at[c, pl.ds(s * RPT, RPT)])


_sc_aggregate = pl.kernel(
    _sc_body,
    out_type=jax.ShapeDtypeStruct((NC, NP, D), jnp.float32),
    mesh=plsc.VectorSubcoreMesh(
        core_axis_name="c", subcore_axis_name="s",
        num_cores=NC, num_subcores=NS),
    scratch_types=[
        pltpu.VMEM((NCHUNK, CHUNK), jnp.int32),
        pltpu.VMEM((NCHUNK, CHUNK), jnp.int32),
        pltpu.VMEM((EPT,), jnp.float32),
        pltpu.VMEM((CHUNK, D), jnp.float32),
        pltpu.VMEM_SHARED((NP, D), jnp.float32),
        pltpu.SemaphoreType.DMA,
    ],
)

_BM = 1024


def _tc_body(p_ref, w_ref, o_ref):
    o_ref[...] = jnp.dot(p_ref[0] + p_ref[1], w_ref[...],
                         preferred_element_type=jnp.float32)


def _tc_matmul(partials, weight):
    return pl.pallas_call(
        _tc_body,
        grid=(NP // _BM,),
        in_specs=[
            pl.BlockSpec((NC, _BM, D), lambda i: (0, i, 0)),
            pl.BlockSpec((D, D), lambda i: (0, 0)),
        ],
        out_specs=pl.BlockSpec((_BM, D), lambda i: (i, 0)),
        out_shape=jax.ShapeDtypeStruct((NP, D), jnp.float32),
    )(partials, weight)


@jax.jit
def _impl(x, edge_index, adj_values, weight):
    row = edge_index[0]
    col = edge_index[1]
    colp = jnp.zeros((E_PAD,), jnp.int32).at[:E].set(col)
    rowp = jnp.zeros((E_PAD,), jnp.int32).at[:E].set(row)
    adjp = jnp.zeros((E_PAD,), jnp.float32).at[:E].set(adj_values)
    colp = colp.reshape(NW, NCHUNK, CHUNK)
    rowp = rowp.reshape(NW, NCHUNK, CHUNK)
    adjp = adjp.reshape(NW, EPT)
    zeros = jnp.zeros((RPT, D), jnp.float32)
    partials = _sc_aggregate(x, colp, rowp, adjp, zeros)
    return _tc_matmul(partials, weight)[:N]


def kernel(x, edge_index, adj_values, weight):
    return _impl(x, edge_index, adj_values, weight)


# speedup vs baseline: 3.0173x; 1.0363x over previous
import jax
import jax.numpy as jnp
from jax import lax
from jax.experimental import pallas as pl
from jax.experimental.pallas import tpu as pltpu
from jax.experimental.pallas import tpu_sc as plsc

N = 10000
NP = 10240
D = 128
E = 320000
NC = 2
NS = 16
NW = NC * NS
EPT = 10240
E_PAD = NW * EPT
CHUNK = 64
SCH = 8
NSB = EPT // (CHUNK * SCH)
RPT = NP // NS

_LANES = 16


def _sc_body(x_hbm, col_hbm, row_hbm, adj_hbm, zero_hbm, out_hbm,
             colv, rowv, adjv, rows_a, rows_b, acc, sem_a, sem_b):
    c = lax.axis_index("c")
    s = lax.axis_index("s")
    wid = s * NC + c

    pltpu.sync_copy(zero_hbm, acc.at[pl.ds(s * RPT, RPT)])
    plsc.subcore_barrier()

    def gather_start(k, buf, sem):
        pltpu.async_copy(x_hbm.at[colv.at[k]], buf, sem)

    def gather_wait(k, buf, sem):
        pltpu.make_async_copy(x_hbm.at[colv.at[k]], buf, sem).wait()

    def scale_chunk(k, rows):
        base = k * CHUNK
        for g in range(CHUNK // _LANES):
            a16 = adjv[pl.ds(base + g * _LANES, _LANES)]
            for j in range(_LANES):
                e = g * _LANES + j
                scale = jnp.full((_LANES,), a16[j], jnp.float32)
                for v in range(D // _LANES):
                    sl = pl.ds(v * _LANES, _LANES)
                    rows[e, sl] = rows[e, sl] * scale

    def sb_body(sb, carry):
        pltpu.sync_copy(col_hbm.at[wid, sb], colv)
        pltpu.sync_copy(row_hbm.at[wid, sb], rowv)
        pltpu.sync_copy(adj_hbm.at[wid, sb], adjv)
        gather_start(0, rows_a, sem_a)

        def pair_body(j, carry2):
            k0 = 2 * j
            k1 = k0 + 1
            gather_wait(k0, rows_a, sem_a)
            gather_start(k1, rows_b, sem_b)
            scale_chunk(k0, rows_a)
            pltpu.sync_copy(rows_a, acc.at[rowv.at[k0]], add=True)
            gather_wait(k1, rows_b, sem_b)

            @pl.when(k0 + 2 < SCH)
            def _():
                gather_start(k0 + 2, rows_a, sem_a)

            scale_chunk(k1, rows_b)
            pltpu.sync_copy(rows_b, acc.at[rowv.at[k1]], add=True)
            return carry2

        lax.fori_loop(0, SCH // 2, pair_body, 0)
        return carry

    lax.fori_loop(0, NSB, sb_body, 0)
    plsc.subcore_barrier()
    pltpu.sync_copy(acc.at[pl.ds(s * RPT, RPT)],
                    out_hbm.at[c, pl.ds(s * RPT, RPT)])


_sc_aggregate = pl.kernel(
    _sc_body,
    out_type=jax.ShapeDtypeStruct((NC, NP, D), jnp.float32),
    mesh=plsc.VectorSubcoreMesh(
        core_axis_name="c", subcore_axis_name="s",
        num_cores=NC, num_subcores=NS),
    scratch_types=[
        pltpu.VMEM((SCH, CHUNK), jnp.int32),
        pltpu.VMEM((SCH, CHUNK), jnp.int32),
        pltpu.VMEM((SCH * CHUNK,), jnp.float32),
        pltpu.VMEM((CHUNK, D), jnp.float32),
        pltpu.VMEM((CHUNK, D), jnp.float32),
        pltpu.VMEM_SHARED((NP, D), jnp.float32),
        pltpu.SemaphoreType.DMA,
        pltpu.SemaphoreType.DMA,
    ],
)

_BM = 1024


def _tc_body(p_ref, w_ref, o_ref):
    o_ref[...] = jnp.dot(p_ref[0] + p_ref[1], w_ref[...],
                         preferred_element_type=jnp.float32)


def _tc_matmul(partials, weight):
    return pl.pallas_call(
        _tc_body,
        grid=(NP // _BM,),
        in_specs=[
            pl.BlockSpec((NC, _BM, D), lambda i: (0, i, 0)),
            pl.BlockSpec((D, D), lambda i: (0, 0)),
        ],
        out_specs=pl.BlockSpec((_BM, D), lambda i: (i, 0)),
        out_shape=jax.ShapeDtypeStruct((NP, D), jnp.float32),
    )(partials, weight)


@jax.jit
def _impl(x, edge_index, adj_values, weight):
    row = edge_index[0]
    col = edge_index[1]
    colp = jnp.zeros((E_PAD,), jnp.int32).at[:E].set(col)
    rowp = jnp.zeros((E_PAD,), jnp.int32).at[:E].set(row)
    adjp = jnp.zeros((E_PAD,), jnp.float32).at[:E].set(adj_values)
    colp = colp.reshape(NW, NSB, SCH, CHUNK)
    rowp = rowp.reshape(NW, NSB, SCH, CHUNK)
    adjp = adjp.reshape(NW, NSB, SCH * CHUNK)
    zeros = jnp.zeros((RPT, D), jnp.float32)
    partials = _sc_aggregate(x, colp, rowp, adjp, zeros)
    return _tc_matmul(partials, weight)[:N]


def kernel(x, edge_index, adj_values, weight):
    return _impl(x, edge_index, adj_values, weight)
